# Initial kernel scaffold; baseline (speedup 1.0000x reference)
import functools

import jax
import jax.numpy as jnp
from jax import lax
from jax.experimental import pallas as pl
from jax.experimental.pallas import tpu as pltpu
from jax.experimental.pallas import tpu_sc as plsc

N, E, D, H = 10000, 320000, 128, 128

NC, NS = 2, 16
CHK = 80
EPC = E // NC
ROWS_PER_TILE = EPC // (NS * CHK)
NPT = N // NS
ZROWS = 25
DEGW = 16


def _agg_body(with_deg, h_hbm, src_hbm, dst_hbm, *refs):
  if with_deg:
    (parts_hbm, deg_hbm, acc, dacc, src_v, dst_v, msgs_v, ones_v, zb, zbd,
     gsem) = refs
  else:
    parts_hbm, acc, src_v, dst_v, msgs_v, zb, gsem = refs

  core = lax.axis_index("c")
  sub = lax.axis_index("s")

  def _zb_init(i, _):
    zb[i // 8, pl.ds((i % 8) * 16, 16)] = jnp.zeros((16,), jnp.float32)
    return 0
  lax.fori_loop(0, ZROWS * (D // 16), _zb_init, 0)

  def _zero_acc(j, _):
    pltpu.sync_copy(zb, acc.at[pl.ds(sub * NPT + j * ZROWS, ZROWS), :])
    return 0
  lax.fori_loop(0, NPT // ZROWS, _zero_acc, 0)

  if with_deg:
    def _zbd_init(i, _):
      zbd[i, pl.ds(0, 16)] = jnp.zeros((16,), jnp.float32)
      return 0
    lax.fori_loop(0, ZROWS, _zbd_init, 0)

    def _zero_dacc(j, _):
      pltpu.sync_copy(zbd, dacc.at[pl.ds(sub * NPT + j * ZROWS, ZROWS), :])
      return 0
    lax.fori_loop(0, NPT // ZROWS, _zero_dacc, 0)

    def _ones_init(i, _):
      ones_v[i, pl.ds(0, 16)] = jnp.ones((16,), jnp.float32)
      return 0
    lax.fori_loop(0, CHK, _ones_init, 0)

  plsc.subcore_barrier()

  row0 = core * (EPC // CHK) + sub * ROWS_PER_TILE
  pltpu.sync_copy(src_hbm.at[pl.ds(row0, ROWS_PER_TILE), :], src_v)
  pltpu.sync_copy(dst_hbm.at[pl.ds(row0, ROWS_PER_TILE), :], dst_v)

  def _chunk(c, _):
    pltpu.async_copy(h_hbm.at[src_v.at[c]], msgs_v, gsem).wait()
    pltpu.sync_copy(msgs_v, acc.at[dst_v.at[c]], add=True)
    if with_deg:
      pltpu.sync_copy(ones_v, dacc.at[dst_v.at[c]], add=True)
    return 0
  lax.fori_loop(0, ROWS_PER_TILE, _chunk, 0)

  plsc.subcore_barrier()

  pltpu.sync_copy(acc.at[pl.ds(sub * NPT, NPT), :],
                  parts_hbm.at[core, pl.ds(sub * NPT, NPT), :])
  if with_deg:
    pltpu.sync_copy(dacc.at[pl.ds(sub * NPT, NPT), :],
                    deg_hbm.at[core, pl.ds(sub * NPT, NPT), :])


def _make_agg(with_deg):
  mesh = plsc.VectorSubcoreMesh(core_axis_name="c", subcore_axis_name="s")
  out_type = [jax.ShapeDtypeStruct((NC, N, D), jnp.float32)]
  scratch = [
      pltpu.VMEM_SHARED((N, D), jnp.float32),
  ]
  if with_deg:
    out_type.append(jax.ShapeDtypeStruct((NC, N, DEGW), jnp.float32))
    scratch.append(pltpu.VMEM_SHARED((N, DEGW), jnp.float32))
  scratch += [
      pltpu.VMEM((ROWS_PER_TILE, CHK), jnp.int32),
      pltpu.VMEM((ROWS_PER_TILE, CHK), jnp.int32),
      pltpu.VMEM((CHK, D), jnp.float32),
  ]
  if with_deg:
    scratch.append(pltpu.VMEM((CHK, DEGW), jnp.float32))
  scratch.append(pltpu.VMEM((ZROWS, D), jnp.float32))
  if with_deg:
    scratch.append(pltpu.VMEM((ZROWS, DEGW), jnp.float32))
  scratch.append(pltpu.SemaphoreType.DMA)

  return pl.kernel(
      functools.partial(_agg_body, with_deg),
      out_type=tuple(out_type) if len(out_type) > 1 else out_type[0],
      mesh=mesh,
      scratch_types=scratch,
  )


_agg_with_deg = _make_agg(True)
_agg_no_deg = _make_agg(False)


def _layer_body(residual, parts_ref, deg_ref, h_ref, wl_ref, bl_ref, wr_ref,
                g_ref, bt_ref, out_ref):
  agg = parts_ref[0] + parts_ref[1]
  deg = deg_ref[0][:, :1] + deg_ref[1][:, :1]
  agg = agg / jnp.maximum(deg, 1.0)
  h = h_ref[...]
  z = lax.dot_general(agg, wl_ref[...], (((1,), (1,)), ((), ())),
                      preferred_element_type=jnp.float32)
  z = z + bl_ref[...]
  z = z + lax.dot_general(h, wr_ref[...], (((1,), (1,)), ((), ())),
                          preferred_element_type=jnp.float32)
  mu = jnp.mean(z, axis=-1, keepdims=True)
  var = jnp.mean((z - mu) ** 2, axis=-1, keepdims=True)
  zn = (z - mu) * lax.rsqrt(var + 1e-5) * g_ref[...] + bt_ref[...]
  if residual:
    zn = zn + h
  out_ref[...] = jnp.maximum(zn, 0.0)


def _make_layer(residual):
  blk = 1000
  grid = N // blk
  return pl.pallas_call(
      functools.partial(_layer_body, residual),
      grid=(grid,),
      in_specs=[
          pl.BlockSpec((NC, blk, D), lambda i: (0, i, 0)),
          pl.BlockSpec((NC, blk, DEGW), lambda i: (0, i, 0)),
          pl.BlockSpec((blk, D), lambda i: (i, 0)),
          pl.BlockSpec((H, D), lambda i: (0, 0)),
          pl.BlockSpec((1, H), lambda i: (0, 0)),
          pl.BlockSpec((H, H), lambda i: (0, 0)),
          pl.BlockSpec((1, H), lambda i: (0, 0)),
          pl.BlockSpec((1, H), lambda i: (0, 0)),
      ],
      out_specs=pl.BlockSpec((blk, D), lambda i: (i, 0)),
      out_shape=jax.ShapeDtypeStruct((N, D), jnp.float32),
  )


_layer1 = _make_layer(False)
_layer2 = _make_layer(True)


@jax.jit
def kernel(x, edge_index, W_l1, b_l1, W_r1, g1, bt1, W_l2, b_l2, W_r2, g2,
           bt2):
  src = edge_index[0].astype(jnp.int32).reshape(E // CHK, CHK)
  dst = edge_index[1].astype(jnp.int32).reshape(E // CHK, CHK)

  parts1, degp = _agg_with_deg(x, src, dst)
  h1 = _layer1(parts1, degp, x, W_l1, b_l1.reshape(1, H), W_r1,
               g1.reshape(1, H), bt1.reshape(1, H))
  parts2 = _agg_no_deg(h1, src, dst)
  h2 = _layer2(parts2, degp, h1, W_l2, b_l2.reshape(1, H), W_r2,
               g2.reshape(1, H), bt2.reshape(1, H))
  return h2


# speedup vs baseline: 4.8092x; 4.8092x over previous
import functools

import jax
import jax.numpy as jnp
from jax import lax
from jax.experimental import pallas as pl
from jax.experimental.pallas import tpu as pltpu
from jax.experimental.pallas import tpu_sc as plsc

N, E, D, H = 10000, 320000, 128, 128

NC, NS = 2, 16
CHK = 80
EPT = E // (NC * NS)
CHUNKS = EPT // CHK
NPT = 624
TAIL = N - NS * NPT
ZROWS = 48
DEGW = 128


def _zero_shared(zb, shared, sub):
  def _zero(j, _):
    pltpu.sync_copy(zb, shared.at[pl.ds(sub * NPT + j * ZROWS, ZROWS), :])
    return 0
  lax.fori_loop(0, NPT // ZROWS, _zero, 0)

  @pl.when(sub == NS - 1)
  def _tail():
    pltpu.sync_copy(zb.at[pl.ds(0, TAIL), :],
                    shared.at[pl.ds(NS * NPT, TAIL), :])


def _writeout_shared(shared, out_hbm, core, sub):
  pltpu.sync_copy(shared.at[pl.ds(sub * NPT, NPT), :],
                  out_hbm.at[core, pl.ds(sub * NPT, NPT), :])

  @pl.when(sub == NS - 1)
  def _tail():
    pltpu.sync_copy(shared.at[pl.ds(NS * NPT, TAIL), :],
                    out_hbm.at[core, pl.ds(NS * NPT, TAIL), :])


def _fill_rows(ref, rows, width, value):
  per_row = width // 16
  def _st(i, _):
    ref[i // per_row, pl.ds((i % per_row) * 16, 16)] = jnp.full(
        (16,), value, jnp.float32)
    return 0
  lax.fori_loop(0, rows * per_row, _st, 0)


def _agg_body(h_hbm, src_hbm, dst_hbm, parts_hbm, acc, src_c, dst_c, msgs_v,
              zb, gsem):
  core = lax.axis_index("c")
  sub = lax.axis_index("s")

  _fill_rows(zb, ZROWS, D, 0.0)
  _zero_shared(zb, acc, sub)
  plsc.subcore_barrier()

  base = (core * NS + sub) * EPT

  def _chunk(c, _):
    off = base + c * CHK
    pltpu.sync_copy(src_hbm.at[pl.ds(off, CHK)], src_c)
    pltpu.sync_copy(dst_hbm.at[pl.ds(off, CHK)], dst_c)
    pltpu.async_copy(h_hbm.at[src_c], msgs_v, gsem).wait()
    pltpu.sync_copy(msgs_v, acc.at[dst_c], add=True)
    return 0
  lax.fori_loop(0, CHUNKS, _chunk, 0)

  plsc.subcore_barrier()
  _writeout_shared(acc, parts_hbm, core, sub)


_agg = pl.kernel(
    _agg_body,
    out_type=jax.ShapeDtypeStruct((NC, N, D), jnp.float32),
    mesh=plsc.VectorSubcoreMesh(core_axis_name="c", subcore_axis_name="s"),
    scratch_types=[
        pltpu.VMEM_SHARED((N, D), jnp.float32),
        pltpu.VMEM((CHK,), jnp.int32),
        pltpu.VMEM((CHK,), jnp.int32),
        pltpu.VMEM((CHK, D), jnp.float32),
        pltpu.VMEM((ZROWS, D), jnp.float32),
        pltpu.SemaphoreType.DMA,
    ],
)


def _deg_body(dst_hbm, deg_hbm, dacc, dst_c, ones_v, zbd):
  core = lax.axis_index("c")
  sub = lax.axis_index("s")

  _fill_rows(zbd, ZROWS, DEGW, 0.0)
  _zero_shared(zbd, dacc, sub)
  _fill_rows(ones_v, CHK, DEGW, 1.0)
  plsc.subcore_barrier()

  base = (core * NS + sub) * EPT

  def _chunk(c, _):
    pltpu.sync_copy(dst_hbm.at[pl.ds(base + c * CHK, CHK)], dst_c)
    pltpu.sync_copy(ones_v, dacc.at[dst_c], add=True)
    return 0
  lax.fori_loop(0, CHUNKS, _chunk, 0)

  plsc.subcore_barrier()
  _writeout_shared(dacc, deg_hbm, core, sub)


_deg = pl.kernel(
    _deg_body,
    out_type=jax.ShapeDtypeStruct((NC, N, DEGW), jnp.float32),
    mesh=plsc.VectorSubcoreMesh(core_axis_name="c", subcore_axis_name="s"),
    scratch_types=[
        pltpu.VMEM_SHARED((N, DEGW), jnp.float32),
        pltpu.VMEM((CHK,), jnp.int32),
        pltpu.VMEM((CHK, DEGW), jnp.float32),
        pltpu.VMEM((ZROWS, DEGW), jnp.float32),
    ],
)


def _layer_body(residual, parts_ref, deg_ref, h_ref, wl_ref, bl_ref, wr_ref,
                g_ref, bt_ref, out_ref):
  agg = parts_ref[0] + parts_ref[1]
  deg = deg_ref[0][:, :1] + deg_ref[1][:, :1]
  agg = agg / jnp.maximum(deg, 1.0)
  h = h_ref[...]
  z = lax.dot_general(agg, wl_ref[...], (((1,), (1,)), ((), ())),
                      preferred_element_type=jnp.float32)
  z = z + bl_ref[...]
  z = z + lax.dot_general(h, wr_ref[...], (((1,), (1,)), ((), ())),
                          preferred_element_type=jnp.float32)
  mu = jnp.mean(z, axis=-1, keepdims=True)
  var = jnp.mean((z - mu) ** 2, axis=-1, keepdims=True)
  zn = (z - mu) * lax.rsqrt(var + 1e-5) * g_ref[...] + bt_ref[...]
  if residual:
    zn = zn + h
  out_ref[...] = jnp.maximum(zn, 0.0)


def _make_layer(residual):
  blk = 1000
  grid = N // blk
  return pl.pallas_call(
      functools.partial(_layer_body, residual),
      grid=(grid,),
      in_specs=[
          pl.BlockSpec((NC, blk, D), lambda i: (0, i, 0)),
          pl.BlockSpec((NC, blk, DEGW), lambda i: (0, i, 0)),
          pl.BlockSpec((blk, D), lambda i: (i, 0)),
          pl.BlockSpec((H, D), lambda i: (0, 0)),
          pl.BlockSpec((1, H), lambda i: (0, 0)),
          pl.BlockSpec((H, H), lambda i: (0, 0)),
          pl.BlockSpec((1, H), lambda i: (0, 0)),
          pl.BlockSpec((1, H), lambda i: (0, 0)),
      ],
      out_specs=pl.BlockSpec((blk, D), lambda i: (i, 0)),
      out_shape=jax.ShapeDtypeStruct((N, D), jnp.float32),
  )


_layer1 = _make_layer(False)
_layer2 = _make_layer(True)


@jax.jit
def kernel(x, edge_index, W_l1, b_l1, W_r1, g1, bt1, W_l2, b_l2, W_r2, g2,
           bt2):
  src = edge_index[0].astype(jnp.int32)
  dst = edge_index[1].astype(jnp.int32)

  degp = _deg(dst)
  parts1 = _agg(x, src, dst)
  h1 = _layer1(parts1, degp, x, W_l1, b_l1.reshape(1, H), W_r1,
               g1.reshape(1, H), bt1.reshape(1, H))
  parts2 = _agg(h1, src, dst)
  h2 = _layer2(parts2, degp, h1, W_l2, b_l2.reshape(1, H), W_r2,
               g2.reshape(1, H), bt2.reshape(1, H))
  return h2


# speedup vs baseline: 10.7714x; 2.2398x over previous
import functools

import jax
import jax.numpy as jnp
from jax import lax
from jax.experimental import pallas as pl
from jax.experimental.pallas import tpu as pltpu
from jax.experimental.pallas import tpu_sc as plsc

N, E, D, H = 10000, 320000, 128, 128

NC, NS = 2, 16
CHK = 125
EPT = E // (NC * NS)
ROWS = EPT // CHK
HALF = ROWS // 2
NBUF = 2
NPT = 624
TAIL = N - NS * NPT
ZROWS = 48
DEGW = 128


def _zero_shared(zb, shared, sub):
  def _zero(j, _):
    pltpu.sync_copy(zb, shared.at[pl.ds(sub * NPT + j * ZROWS, ZROWS), :])
    return 0
  lax.fori_loop(0, NPT // ZROWS, _zero, 0)

  @pl.when(sub == NS - 1)
  def _tail():
    pltpu.sync_copy(zb.at[pl.ds(0, TAIL), :],
                    shared.at[pl.ds(NS * NPT, TAIL), :])


def _writeout_shared(shared, out_hbm, core, sub):
  pltpu.sync_copy(shared.at[pl.ds(sub * NPT, NPT), :],
                  out_hbm.at[core, pl.ds(sub * NPT, NPT), :])

  @pl.when(sub == NS - 1)
  def _tail():
    pltpu.sync_copy(shared.at[pl.ds(NS * NPT, TAIL), :],
                    out_hbm.at[core, pl.ds(NS * NPT, TAIL), :])


def _fill_rows(ref, rows, width, value):
  per_row = width // 16
  def _st(i, _):
    ref[i // per_row, pl.ds((i % per_row) * 16, 16)] = jnp.full(
        (16,), value, jnp.float32)
    return 0
  lax.fori_loop(0, rows * per_row, _st, 0)


def _agg_body(h_hbm, src_hbm, dst_hbm, parts_hbm, acc, src_v, dst_v,
              m0, m1, s0, s1):
  msgs = (m0, m1)
  sems = (s0, s1)
  core = lax.axis_index("c")
  sub = lax.axis_index("s")

  _fill_rows(m0, ZROWS, D, 0.0)
  _zero_shared(m0.at[pl.ds(0, ZROWS), :], acc, sub)
  plsc.subcore_barrier()

  row0 = (core * NS + sub) * ROWS
  for half in range(2):
    pltpu.sync_copy(src_hbm.at[pl.ds(row0 + half * HALF, HALF), :], src_v)
    pltpu.sync_copy(dst_hbm.at[pl.ds(row0 + half * HALF, HALF), :], dst_v)

    for b in range(NBUF):
      pltpu.async_copy(h_hbm.at[src_v.at[b]], msgs[b], sems[b])

    def _group(g, _):
      for b in range(NBUF):
        c = g * NBUF + b
        pltpu.make_async_copy(h_hbm.at[src_v.at[c]], msgs[b], sems[b]).wait()
        pltpu.sync_copy(msgs[b], acc.at[dst_v.at[c]], add=True)

        @pl.when(c + NBUF < HALF)
        def _refill():
          pltpu.async_copy(h_hbm.at[src_v.at[c + NBUF]], msgs[b], sems[b])
      return 0
    lax.fori_loop(0, HALF // NBUF, _group, 0)

  plsc.subcore_barrier()
  _writeout_shared(acc, parts_hbm, core, sub)


_agg = pl.kernel(
    _agg_body,
    out_type=jax.ShapeDtypeStruct((NC, N, D), jnp.float32),
    mesh=plsc.VectorSubcoreMesh(core_axis_name="c", subcore_axis_name="s"),
    scratch_types=[
        pltpu.VMEM_SHARED((N, D), jnp.float32),
        pltpu.VMEM((HALF, CHK), jnp.int32),
        pltpu.VMEM((HALF, CHK), jnp.int32),
    ] + [pltpu.VMEM((CHK, D), jnp.float32)] * NBUF
    + [pltpu.SemaphoreType.DMA] * NBUF,
)


def _deg_body(dst_hbm, deg_hbm, dacc, dst_v, ones_v, zbd, sem):
  core = lax.axis_index("c")
  sub = lax.axis_index("s")

  _fill_rows(zbd, ZROWS, DEGW, 0.0)
  _zero_shared(zbd, dacc, sub)
  _fill_rows(ones_v, CHK, DEGW, 1.0)
  plsc.subcore_barrier()

  row0 = (core * NS + sub) * ROWS
  pltpu.sync_copy(dst_hbm.at[pl.ds(row0, ROWS), :], dst_v)

  def _group(g, _):
    for b in range(NBUF):
      pltpu.async_copy(ones_v, dacc.at[dst_v.at[g * NBUF + b]], sem,
                       add=True)
    for b in range(NBUF):
      pltpu.make_async_copy(ones_v, dacc.at[dst_v.at[g * NBUF + b]],
                            sem).wait()
    return 0
  lax.fori_loop(0, ROWS // NBUF, _group, 0)

  plsc.subcore_barrier()
  _writeout_shared(dacc, deg_hbm, core, sub)


_deg = pl.kernel(
    _deg_body,
    out_type=jax.ShapeDtypeStruct((NC, N, DEGW), jnp.float32),
    mesh=plsc.VectorSubcoreMesh(core_axis_name="c", subcore_axis_name="s"),
    scratch_types=[
        pltpu.VMEM_SHARED((N, DEGW), jnp.float32),
        pltpu.VMEM((ROWS, CHK), jnp.int32),
        pltpu.VMEM((CHK, DEGW), jnp.float32),
        pltpu.VMEM((ZROWS, DEGW), jnp.float32),
        pltpu.SemaphoreType.DMA,
    ],
)


def _layer_body(residual, parts_ref, deg_ref, h_ref, wl_ref, bl_ref, wr_ref,
                g_ref, bt_ref, out_ref):
  agg = parts_ref[0] + parts_ref[1]
  deg = deg_ref[0][:, :1] + deg_ref[1][:, :1]
  agg = agg / jnp.maximum(deg, 1.0)
  h = h_ref[...]
  z = lax.dot_general(agg, wl_ref[...], (((1,), (1,)), ((), ())),
                      preferred_element_type=jnp.float32)
  z = z + bl_ref[...]
  z = z + lax.dot_general(h, wr_ref[...], (((1,), (1,)), ((), ())),
                          preferred_element_type=jnp.float32)
  mu = jnp.mean(z, axis=-1, keepdims=True)
  var = jnp.mean((z - mu) ** 2, axis=-1, keepdims=True)
  zn = (z - mu) * lax.rsqrt(var + 1e-5) * g_ref[...] + bt_ref[...]
  if residual:
    zn = zn + h
  out_ref[...] = jnp.maximum(zn, 0.0)


def _make_layer(residual):
  blk = 1000
  grid = N // blk
  return pl.pallas_call(
      functools.partial(_layer_body, residual),
      grid=(grid,),
      in_specs=[
          pl.BlockSpec((NC, blk, D), lambda i: (0, i, 0)),
          pl.BlockSpec((NC, blk, DEGW), lambda i: (0, i, 0)),
          pl.BlockSpec((blk, D), lambda i: (i, 0)),
          pl.BlockSpec((H, D), lambda i: (0, 0)),
          pl.BlockSpec((1, H), lambda i: (0, 0)),
          pl.BlockSpec((H, H), lambda i: (0, 0)),
          pl.BlockSpec((1, H), lambda i: (0, 0)),
          pl.BlockSpec((1, H), lambda i: (0, 0)),
      ],
      out_specs=pl.BlockSpec((blk, D), lambda i: (i, 0)),
      out_shape=jax.ShapeDtypeStruct((N, D), jnp.float32),
  )


_layer1 = _make_layer(False)
_layer2 = _make_layer(True)


@jax.jit
def kernel(x, edge_index, W_l1, b_l1, W_r1, g1, bt1, W_l2, b_l2, W_r2, g2,
           bt2):
  src = edge_index[0].astype(jnp.int32).reshape(E // CHK, CHK)
  dst = edge_index[1].astype(jnp.int32).reshape(E // CHK, CHK)

  degp = _deg(dst)
  parts1 = _agg(x, src, dst)
  h1 = _layer1(parts1, degp, x, W_l1, b_l1.reshape(1, H), W_r1,
               g1.reshape(1, H), bt1.reshape(1, H))
  parts2 = _agg(h1, src, dst)
  h2 = _layer2(parts2, degp, h1, W_l2, b_l2.reshape(1, H), W_r2,
               g2.reshape(1, H), bt2.reshape(1, H))
  return h2


# speedup vs baseline: 10.8024x; 1.0029x over previous
import functools

import jax
import jax.numpy as jnp
from jax import lax
from jax.experimental import pallas as pl
from jax.experimental.pallas import tpu as pltpu
from jax.experimental.pallas import tpu_sc as plsc

N, E, D, H = 10000, 320000, 128, 128

NC, NS = 2, 16
CHK = 125
EPT = E // (NC * NS)
ROWS = EPT // CHK
HALF = ROWS // 2
NBUF = 2
NPT = 624
TAIL = N - NS * NPT
ZROWS = 48
DEGW = 128
DEG_FIRE = 8


def _zero_shared(zb, shared, sub):
  def _zero(j, _):
    pltpu.sync_copy(zb, shared.at[pl.ds(sub * NPT + j * ZROWS, ZROWS), :])
    return 0
  lax.fori_loop(0, NPT // ZROWS, _zero, 0)

  @pl.when(sub == NS - 1)
  def _tail():
    pltpu.sync_copy(zb.at[pl.ds(0, TAIL), :],
                    shared.at[pl.ds(NS * NPT, TAIL), :])


def _writeout_shared(shared, out_hbm, core, sub):
  pltpu.sync_copy(shared.at[pl.ds(sub * NPT, NPT), :],
                  out_hbm.at[core, pl.ds(sub * NPT, NPT), :])

  @pl.when(sub == NS - 1)
  def _tail():
    pltpu.sync_copy(shared.at[pl.ds(NS * NPT, TAIL), :],
                    out_hbm.at[core, pl.ds(NS * NPT, TAIL), :])


def _fill_rows(ref, rows, width, value):
  per_row = width // 16
  def _st(i, _):
    ref[i // per_row, pl.ds((i % per_row) * 16, 16)] = jnp.full(
        (16,), value, jnp.float32)
    return 0
  lax.fori_loop(0, rows * per_row, _st, 0)


def _agg_body(h_hbm, src_hbm, dst_hbm, parts_hbm, acc, src_v, dst_v,
              m0, m1, s0, s1):
  msgs = (m0, m1)
  sems = (s0, s1)
  core = lax.axis_index("c")
  sub = lax.axis_index("s")

  _fill_rows(m0, ZROWS, D, 0.0)
  _zero_shared(m0.at[pl.ds(0, ZROWS), :], acc, sub)
  plsc.subcore_barrier()

  row0 = (core * NS + sub) * ROWS
  for half in range(ROWS // HALF):
    pltpu.sync_copy(src_hbm.at[pl.ds(row0 + half * HALF, HALF), :], src_v)
    pltpu.sync_copy(dst_hbm.at[pl.ds(row0 + half * HALF, HALF), :], dst_v)

    for b in range(NBUF):
      pltpu.async_copy(h_hbm.at[src_v.at[b]], msgs[b], sems[b])

    def _group(g, _):
      for b in range(NBUF):
        c = g * NBUF + b
        pltpu.make_async_copy(h_hbm.at[src_v.at[c]], msgs[b], sems[b]).wait()
        pltpu.sync_copy(msgs[b], acc.at[dst_v.at[c]], add=True)

        @pl.when(c + NBUF < HALF)
        def _refill():
          pltpu.async_copy(h_hbm.at[src_v.at[c + NBUF]], msgs[b], sems[b])
      return 0
    lax.fori_loop(0, HALF // NBUF, _group, 0)

  plsc.subcore_barrier()
  _writeout_shared(acc, parts_hbm, core, sub)


_agg = pl.kernel(
    _agg_body,
    out_type=jax.ShapeDtypeStruct((NC, N, D), jnp.float32),
    mesh=plsc.VectorSubcoreMesh(core_axis_name="c", subcore_axis_name="s"),
    scratch_types=[
        pltpu.VMEM_SHARED((N, D), jnp.float32),
        pltpu.VMEM((HALF, CHK), jnp.int32),
        pltpu.VMEM((HALF, CHK), jnp.int32),
    ] + [pltpu.VMEM((CHK, D), jnp.float32)] * NBUF
    + [pltpu.SemaphoreType.DMA] * NBUF,
)


def _deg_body(dst_hbm, deg_hbm, dacc, dst_v, ones_v, zbd, sem):
  core = lax.axis_index("c")
  sub = lax.axis_index("s")

  _fill_rows(zbd, ZROWS, DEGW, 0.0)
  _zero_shared(zbd, dacc, sub)
  _fill_rows(ones_v, CHK, DEGW, 1.0)
  plsc.subcore_barrier()

  row0 = (core * NS + sub) * ROWS
  pltpu.sync_copy(dst_hbm.at[pl.ds(row0, ROWS), :], dst_v)

  def _group(g, _):
    for b in range(DEG_FIRE):
      pltpu.async_copy(ones_v, dacc.at[dst_v.at[g * DEG_FIRE + b]], sem,
                       add=True)
    for b in range(DEG_FIRE):
      pltpu.make_async_copy(ones_v, dacc.at[dst_v.at[g * DEG_FIRE + b]],
                            sem).wait()
    return 0
  lax.fori_loop(0, ROWS // DEG_FIRE, _group, 0)

  plsc.subcore_barrier()
  _writeout_shared(dacc, deg_hbm, core, sub)


_deg = pl.kernel(
    _deg_body,
    out_type=jax.ShapeDtypeStruct((NC, N, DEGW), jnp.float32),
    mesh=plsc.VectorSubcoreMesh(core_axis_name="c", subcore_axis_name="s"),
    scratch_types=[
        pltpu.VMEM_SHARED((N, DEGW), jnp.float32),
        pltpu.VMEM((ROWS, CHK), jnp.int32),
        pltpu.VMEM((CHK, DEGW), jnp.float32),
        pltpu.VMEM((ZROWS, DEGW), jnp.float32),
        pltpu.SemaphoreType.DMA,
    ],
)


def _layer_body(residual, parts_ref, deg_ref, h_ref, wl_ref, bl_ref, wr_ref,
                g_ref, bt_ref, out_ref):
  agg = parts_ref[0] + parts_ref[1]
  deg = deg_ref[0][:, :1] + deg_ref[1][:, :1]
  agg = agg / jnp.maximum(deg, 1.0)
  h = h_ref[...]
  z = lax.dot_general(agg, wl_ref[...], (((1,), (1,)), ((), ())),
                      preferred_element_type=jnp.float32)
  z = z + bl_ref[...]
  z = z + lax.dot_general(h, wr_ref[...], (((1,), (1,)), ((), ())),
                          preferred_element_type=jnp.float32)
  mu = jnp.mean(z, axis=-1, keepdims=True)
  var = jnp.mean((z - mu) ** 2, axis=-1, keepdims=True)
  zn = (z - mu) * lax.rsqrt(var + 1e-5) * g_ref[...] + bt_ref[...]
  if residual:
    zn = zn + h
  out_ref[...] = jnp.maximum(zn, 0.0)


def _make_layer(residual):
  blk = 1000
  grid = N // blk
  return pl.pallas_call(
      functools.partial(_layer_body, residual),
      grid=(grid,),
      in_specs=[
          pl.BlockSpec((NC, blk, D), lambda i: (0, i, 0)),
          pl.BlockSpec((NC, blk, DEGW), lambda i: (0, i, 0)),
          pl.BlockSpec((blk, D), lambda i: (i, 0)),
          pl.BlockSpec((H, D), lambda i: (0, 0)),
          pl.BlockSpec((1, H), lambda i: (0, 0)),
          pl.BlockSpec((H, H), lambda i: (0, 0)),
          pl.BlockSpec((1, H), lambda i: (0, 0)),
          pl.BlockSpec((1, H), lambda i: (0, 0)),
      ],
      out_specs=pl.BlockSpec((blk, D), lambda i: (i, 0)),
      out_shape=jax.ShapeDtypeStruct((N, D), jnp.float32),
  )


_layer1 = _make_layer(False)
_layer2 = _make_layer(True)


@jax.jit
def kernel(x, edge_index, W_l1, b_l1, W_r1, g1, bt1, W_l2, b_l2, W_r2, g2,
           bt2):
  src = edge_index[0].astype(jnp.int32).reshape(E // CHK, CHK)
  dst = edge_index[1].astype(jnp.int32).reshape(E // CHK, CHK)

  degp = _deg(dst)
  parts1 = _agg(x, src, dst)
  h1 = _layer1(parts1, degp, x, W_l1, b_l1.reshape(1, H), W_r1,
               g1.reshape(1, H), bt1.reshape(1, H))
  parts2 = _agg(h1, src, dst)
  h2 = _layer2(parts2, degp, h1, W_l2, b_l2.reshape(1, H), W_r2,
               g2.reshape(1, H), bt2.reshape(1, H))
  return h2
